# baseline (speedup 1.0000x reference)
import functools

import jax
import jax.numpy as jnp
from jax import lax
from jax.experimental import pallas as pl
from jax.experimental.pallas import tpu as pltpu
from jax.experimental.pallas import tpu_sc as plsc

D_MODEL = 64
SCALE = 8.0
NW = 32
L = 16
XROWS = 4
SPLITS = ((0, 104), (104, 96))
RC = XROWS * 200


def _build(b0, b1):
    rows_per_w = b0 // NW
    n_chunks = rows_per_w // XROWS
    mesh = plsc.VectorSubcoreMesh(core_axis_name="c", subcore_axis_name="s")

    @functools.partial(
        pl.kernel,
        mesh=mesh,
        compiler_params=pltpu.CompilerParams(use_tc_tiling_on_sc=False),
        out_type=jax.ShapeDtypeStruct((b0, b1, D_MODEL), jnp.float32),
        scratch_types=[
            pltpu.VMEM((rows_per_w, b1), jnp.int32),
            pltpu.VMEM((2, RC, D_MODEL), jnp.float32),
            pltpu.SemaphoreType.DMA,
            pltpu.SemaphoreType.DMA,
            pltpu.SemaphoreType.DMA,
        ],
    )
    def emb_kernel(x_hbm, lut_hbm, out_hbm, idx_v, bufs, gsem0, gsem1, osem):
        wid = lax.axis_index("s") * 2 + lax.axis_index("c")
        row0 = wid * rows_per_w
        gsems = (gsem0, gsem1)
        pltpu.sync_copy(x_hbm.at[pl.ds(row0, rows_per_w)], idx_v)

        def fire_gathers(s, b):
            for r in range(XROWS):
                for off, sz in SPLITS:
                    pltpu.async_copy(
                        lut_hbm.at[idx_v.at[s * XROWS + r, pl.ds(off, sz)]],
                        bufs.at[b, pl.ds(r * 200 + off, sz)],
                        gsems[b],
                    )

        def drain_gathers(b):
            pltpu.make_async_copy(
                lut_hbm.at[pl.ds(0, RC)], bufs.at[b], gsems[b]
            ).wait()

        def fire_writes(s, b):
            for r in range(XROWS):
                pltpu.async_copy(
                    bufs.at[b, pl.ds(r * 200, 200)],
                    out_hbm.at[row0 + s * XROWS + r],
                    osem,
                )

        def drain_writes():
            pltpu.make_async_copy(lut_hbm.at[pl.ds(0, RC)], bufs.at[0], osem).wait()

        def scale(b):
            def body(i, c):
                r = i * 4
                for dr in range(4):
                    for cc in range(D_MODEL // L):
                        sl = pl.ds(cc * L, L)
                        bufs[b, r + dr, sl] = bufs[b, r + dr, sl] * SCALE
                return c

            lax.fori_loop(0, RC // 4, body, 0)

        fire_gathers(0, 0)
        n_pairs = n_chunks // 2

        def pair_body(t, carry):
            s0 = 2 * t

            @pl.when(t > 0)
            def _():
                drain_writes()

            fire_gathers(s0 + 1, 1)
            drain_gathers(0)
            scale(0)
            fire_writes(s0, 0)

            @pl.when(t < n_pairs - 1)
            def _():
                drain_writes()
                fire_gathers(s0 + 2, 0)

            drain_gathers(1)
            scale(1)
            fire_writes(s0 + 1, 1)
            return carry

        lax.fori_loop(0, n_pairs, pair_body, 0)
        drain_writes()
        drain_writes()

    return emb_kernel


def kernel(x, lut):
    b0, b1 = x.shape
    out = _build(b0, b1)(x.astype(jnp.int32), lut)
    return out
